# Initial kernel scaffold; baseline (speedup 1.0000x reference)
import jax
import jax.numpy as jnp
from jax.experimental import pallas as pl


def kernel(h, m, rbf, id_j, W_rbf, W1, Wr0a, Wr0b, Wr1a, Wr1b, Wr2a, Wr2b):
    raise NotImplementedError("write your pallas kernel here")


# speedup vs baseline: 1.3513x; 1.3513x over previous
import functools

import jax
import jax.numpy as jnp
import numpy as np
from jax import lax
from jax.experimental import pallas as pl
from jax.experimental.pallas import tpu as pltpu
from jax.experimental.pallas import tpu_sc as plsc

N_ATOMS = 10000
N_EDGES = 160000
D_ATOM = 512
D_EDGE = 256
D_RBF = 16
DH = D_EDGE // 2
INV_SQRT2 = np.float32(1.0 / np.sqrt(2.0))

NS = 16
EPT = N_EDGES // NS
CHUNK = 80
NCHUNK = EPT // CHUNK
ROW_STRIDE = 624
ROW_WIN = 640

EDGE_BLK = 256
ATOM_BLK = 1000


def _silu(x):
    return x / (1.0 + jnp.exp(-x))


def _edge_body(m_ref, rbf_ref, wrbf_ref, o_ref):
    y = m_ref[...] * jnp.dot(rbf_ref[...], wrbf_ref[...],
                             preferred_element_type=jnp.float32)
    o_ref[0] = y[:, :DH]
    o_ref[1] = y[:, DH:]


def _edge_stage(m, rbf, W_rbf):
    grid = (N_EDGES // EDGE_BLK,)
    return pl.pallas_call(
        _edge_body,
        grid=grid,
        in_specs=[
            pl.BlockSpec((EDGE_BLK, D_EDGE), lambda i: (i, 0)),
            pl.BlockSpec((EDGE_BLK, D_RBF), lambda i: (i, 0)),
            pl.BlockSpec((D_RBF, D_EDGE), lambda i: (0, 0)),
        ],
        out_specs=pl.BlockSpec((2, EDGE_BLK, DH), lambda i: (0, i, 0)),
        out_shape=jax.ShapeDtypeStruct((2, N_EDGES, DH), jnp.float32),
    )(m, rbf, W_rbf)


def _seg_body(x_hbm, idx_hbm, z_hbm, out_hbm, idx_ref, rows_ref, acc_ref):
    c = lax.axis_index("c")
    s = lax.axis_index("s")
    pltpu.sync_copy(z_hbm, acc_ref.at[pl.ds(s * ROW_STRIDE, ROW_WIN)])
    pltpu.sync_copy(idx_hbm.at[s], idx_ref)
    plsc.subcore_barrier()

    def body(j, carry):
        e0 = s * EPT + j * CHUNK
        pltpu.sync_copy(x_hbm.at[c, pl.ds(e0, CHUNK)], rows_ref)
        pltpu.sync_copy(rows_ref, acc_ref.at[idx_ref.at[j]], add=True)
        return carry

    lax.fori_loop(0, NCHUNK, body, 0)
    plsc.subcore_barrier()
    pltpu.sync_copy(acc_ref.at[pl.ds(s * ROW_STRIDE, ROW_WIN)],
                    out_hbm.at[c, pl.ds(s * ROW_STRIDE, ROW_WIN)])


def _seg_stage(x_split, idj2d, zeros):
    mesh = plsc.VectorSubcoreMesh(core_axis_name="c", subcore_axis_name="s")
    fn = pl.kernel(
        _seg_body,
        out_type=jax.ShapeDtypeStruct((2, N_ATOMS, DH), jnp.float32),
        mesh=mesh,
        scratch_types=[
            pltpu.VMEM((NCHUNK, CHUNK), jnp.int32),
            pltpu.VMEM((CHUNK, DH), jnp.float32),
            pltpu.VMEM_SHARED((N_ATOMS, DH), jnp.float32),
        ],
    )
    return fn(x_split, idj2d, zeros)


def _mlp_body(x2_ref, w1_ref, wa0, wb0, wa1, wb1, wa2, wb2, o_ref):
    x2 = jnp.concatenate([x2_ref[0], x2_ref[1]], axis=-1)
    x = _silu(jnp.dot(x2, w1_ref[...], preferred_element_type=jnp.float32))
    for wa, wb in ((wa0, wb0), (wa1, wb1), (wa2, wb2)):
        y = _silu(jnp.dot(x, wa[...], preferred_element_type=jnp.float32))
        y = _silu(jnp.dot(y, wb[...], preferred_element_type=jnp.float32))
        x = (x + y) * INV_SQRT2
    o_ref[...] = x


def _mlp_stage(x2_split, W1, Wr0a, Wr0b, Wr1a, Wr1b, Wr2a, Wr2b):
    grid = (N_ATOMS // ATOM_BLK,)
    wspec = pl.BlockSpec((D_ATOM, D_ATOM), lambda i: (0, 0))
    return pl.pallas_call(
        _mlp_body,
        grid=grid,
        in_specs=[
            pl.BlockSpec((2, ATOM_BLK, DH), lambda i: (0, i, 0)),
            pl.BlockSpec((D_EDGE, D_ATOM), lambda i: (0, 0)),
            wspec, wspec, wspec, wspec, wspec, wspec,
        ],
        out_specs=pl.BlockSpec((ATOM_BLK, D_ATOM), lambda i: (i, 0)),
        out_shape=jax.ShapeDtypeStruct((N_ATOMS, D_ATOM), jnp.float32),
    )(x2_split, W1, Wr0a, Wr0b, Wr1a, Wr1b, Wr2a, Wr2b)


def kernel(h, m, rbf, id_j, W_rbf, W1, Wr0a, Wr0b, Wr1a, Wr1b, Wr2a, Wr2b):
    del h
    x_split = _edge_stage(m, rbf, W_rbf)
    idj3d = id_j.astype(jnp.int32).reshape(NS, NCHUNK, CHUNK)
    zeros = jnp.zeros((ROW_WIN, DH), jnp.float32)
    x2_split = _seg_stage(x_split, idj3d, zeros)
    return _mlp_stage(x2_split, W1, Wr0a, Wr0b, Wr1a, Wr1b, Wr2a, Wr2b)


# speedup vs baseline: 2.1752x; 1.6097x over previous
import functools

import jax
import jax.numpy as jnp
import numpy as np
from jax import lax
from jax.experimental import pallas as pl
from jax.experimental.pallas import tpu as pltpu
from jax.experimental.pallas import tpu_sc as plsc

N_ATOMS = 10000
N_EDGES = 160000
D_ATOM = 512
D_EDGE = 256
D_RBF = 16
DH = D_EDGE // 2
INV_SQRT2 = np.float32(1.0 / np.sqrt(2.0))

NS = 16
EPT = N_EDGES // NS
CHUNK = 80
NCHUNK = EPT // CHUNK
ROW_STRIDE = 624
ROW_WIN = 640

EDGE_BLK = 640
ATOM_BLK = 1000


def _silu(x):
    return x / (1.0 + jnp.exp(-x))


def _edge_body(m_ref, rbf_ref, wrbf_ref, o_ref):
    y = m_ref[...] * jnp.dot(rbf_ref[...], wrbf_ref[...],
                             preferred_element_type=jnp.float32)
    o_ref[0] = y[:, :DH]
    o_ref[1] = y[:, DH:]


def _edge_stage(m, rbf, W_rbf):
    grid = (N_EDGES // EDGE_BLK,)
    return pl.pallas_call(
        _edge_body,
        grid=grid,
        in_specs=[
            pl.BlockSpec((EDGE_BLK, D_EDGE), lambda i: (i, 0)),
            pl.BlockSpec((EDGE_BLK, D_RBF), lambda i: (i, 0)),
            pl.BlockSpec((D_RBF, D_EDGE), lambda i: (0, 0)),
        ],
        out_specs=pl.BlockSpec((2, EDGE_BLK, DH), lambda i: (0, i, 0)),
        out_shape=jax.ShapeDtypeStruct((2, N_EDGES, DH), jnp.float32),
    )(m, rbf, W_rbf)


def _seg_body(x_hbm, idx_hbm, z_hbm, out_hbm, idxb_ref, rows_ref, acc_ref,
              gsA, gsB, ssA, ssB):
    c = lax.axis_index("c")
    s = lax.axis_index("s")
    pltpu.sync_copy(z_hbm, acc_ref.at[pl.ds(s * ROW_STRIDE, ROW_WIN)])

    def gather(grp, buf, gsem):
        e0 = s * EPT + grp * CHUNK
        pltpu.async_copy(x_hbm.at[c, pl.ds(e0, CHUNK)],
                         rows_ref.at[buf], gsem)
        pltpu.async_copy(idx_hbm.at[s, grp], idxb_ref.at[buf], gsem)

    def run_group(grp, buf, gsem, ssem):
        pltpu.make_async_copy(x_hbm.at[c, pl.ds(s * EPT, CHUNK)],
                              rows_ref.at[buf], gsem).wait()
        pltpu.make_async_copy(idx_hbm.at[s, 0], idxb_ref.at[buf], gsem).wait()
        pltpu.async_copy(rows_ref.at[buf],
                         acc_ref.at[idxb_ref.at[buf, 0]], ssem, add=True)
        pltpu.make_async_copy(rows_ref.at[buf],
                              acc_ref.at[idxb_ref.at[buf, 0]], ssem).wait()
        @pl.when(grp + 2 < NCHUNK)
        def _():
            gather(grp + 2, buf, gsem)

    gather(0, 0, gsA)
    gather(1, 1, gsB)
    plsc.subcore_barrier()

    def body(g2, carry):
        run_group(2 * g2, 0, gsA, ssA)

        @pl.when(2 * g2 + 1 < NCHUNK)
        def _():
            run_group(2 * g2 + 1, 1, gsB, ssB)

        return carry

    lax.fori_loop(0, (NCHUNK + 1) // 2, body, 0)
    plsc.subcore_barrier()
    pltpu.sync_copy(acc_ref.at[pl.ds(s * ROW_STRIDE, ROW_WIN)],
                    out_hbm.at[c, pl.ds(s * ROW_STRIDE, ROW_WIN)])


def _seg_stage(x_split, idj2d, zeros):
    mesh = plsc.VectorSubcoreMesh(core_axis_name="c", subcore_axis_name="s")
    fn = pl.kernel(
        _seg_body,
        out_type=jax.ShapeDtypeStruct((2, N_ATOMS, DH), jnp.float32),
        mesh=mesh,
        scratch_types=[
            pltpu.VMEM((2, 1, CHUNK), jnp.int32),
            pltpu.VMEM((2, CHUNK, DH), jnp.float32),
            pltpu.VMEM_SHARED((N_ATOMS, DH), jnp.float32),
            pltpu.SemaphoreType.DMA,
            pltpu.SemaphoreType.DMA,
            pltpu.SemaphoreType.DMA,
            pltpu.SemaphoreType.DMA,
        ],
    )
    return fn(x_split, idj2d, zeros)


def _mlp_body(x2_ref, w1_ref, wa0, wb0, wa1, wb1, wa2, wb2, o_ref):
    x2 = jnp.concatenate([x2_ref[0], x2_ref[1]], axis=-1)
    x = _silu(jnp.dot(x2, w1_ref[...], preferred_element_type=jnp.float32))
    for wa, wb in ((wa0, wb0), (wa1, wb1), (wa2, wb2)):
        y = _silu(jnp.dot(x, wa[...], preferred_element_type=jnp.float32))
        y = _silu(jnp.dot(y, wb[...], preferred_element_type=jnp.float32))
        x = (x + y) * INV_SQRT2
    o_ref[...] = x


def _mlp_stage(x2_split, W1, Wr0a, Wr0b, Wr1a, Wr1b, Wr2a, Wr2b):
    grid = (N_ATOMS // ATOM_BLK,)
    wspec = pl.BlockSpec((D_ATOM, D_ATOM), lambda i: (0, 0))
    return pl.pallas_call(
        _mlp_body,
        grid=grid,
        in_specs=[
            pl.BlockSpec((2, ATOM_BLK, DH), lambda i: (0, i, 0)),
            pl.BlockSpec((D_EDGE, D_ATOM), lambda i: (0, 0)),
            wspec, wspec, wspec, wspec, wspec, wspec,
        ],
        out_specs=pl.BlockSpec((ATOM_BLK, D_ATOM), lambda i: (i, 0)),
        out_shape=jax.ShapeDtypeStruct((N_ATOMS, D_ATOM), jnp.float32),
    )(x2_split, W1, Wr0a, Wr0b, Wr1a, Wr1b, Wr2a, Wr2b)


def kernel(h, m, rbf, id_j, W_rbf, W1, Wr0a, Wr0b, Wr1a, Wr1b, Wr2a, Wr2b):
    del h
    x_split = _edge_stage(m, rbf, W_rbf)
    idj4d = id_j.astype(jnp.int32).reshape(NS, NCHUNK, 1, CHUNK)
    zeros = jnp.zeros((ROW_WIN, DH), jnp.float32)
    x2_split = _seg_stage(x_split, idj4d, zeros)
    return _mlp_stage(x2_split, W1, Wr0a, Wr0b, Wr1a, Wr1b, Wr2a, Wr2b)


# speedup vs baseline: 2.2690x; 1.0431x over previous
import functools

import jax
import jax.numpy as jnp
import numpy as np
from jax import lax
from jax.experimental import pallas as pl
from jax.experimental.pallas import tpu as pltpu
from jax.experimental.pallas import tpu_sc as plsc

N_ATOMS = 10000
N_EDGES = 160000
D_ATOM = 512
D_EDGE = 256
D_RBF = 16
DH = D_EDGE // 2
INV_SQRT2 = np.float32(1.0 / np.sqrt(2.0))

NS = 16
EPT = N_EDGES // NS
CHUNK = 80
NCHUNK = EPT // CHUNK
ROW_STRIDE = 624
ROW_WIN = 640

EDGE_BLK = 640
ATOM_BLK = 1000


def _silu(x):
    return x / (1.0 + jnp.exp(-x))


def _edge_body(m_ref, rbf_ref, wrbf_ref, o_ref):
    y = m_ref[...] * jnp.dot(rbf_ref[...], wrbf_ref[...],
                             preferred_element_type=jnp.float32)
    o_ref[0] = y[:, :DH]
    o_ref[1] = y[:, DH:]


def _edge_stage(m, rbf, W_rbf):
    grid = (N_EDGES // EDGE_BLK,)
    return pl.pallas_call(
        _edge_body,
        grid=grid,
        in_specs=[
            pl.BlockSpec((EDGE_BLK, D_EDGE), lambda i: (i, 0)),
            pl.BlockSpec((EDGE_BLK, D_RBF), lambda i: (i, 0)),
            pl.BlockSpec((D_RBF, D_EDGE), lambda i: (0, 0)),
        ],
        out_specs=pl.BlockSpec((2, EDGE_BLK, DH), lambda i: (0, i, 0)),
        out_shape=jax.ShapeDtypeStruct((2, N_EDGES, DH), jnp.float32),
    )(m, rbf, W_rbf)


def _seg_body(x_hbm, idx_hbm, z_hbm, out_hbm, idxb_ref, rows_ref, acc_ref,
              gs0, gs1, gs2, ss0, ss1, ss2):
    c = lax.axis_index("c")
    s = lax.axis_index("s")
    gs = (gs0, gs1, gs2)
    ss = (ss0, ss1, ss2)
    pltpu.sync_copy(z_hbm, acc_ref.at[pl.ds(s * ROW_STRIDE, ROW_WIN)])

    def gather(grp, buf):
        e0 = s * EPT + grp * CHUNK
        pltpu.async_copy(x_hbm.at[c, pl.ds(e0, CHUNK)],
                         rows_ref.at[buf], gs[buf])
        pltpu.async_copy(idx_hbm.at[s, grp], idxb_ref.at[buf], gs[buf])

    def step(g, buf):
        b2 = (buf + 2) % 3
        pltpu.make_async_copy(x_hbm.at[c, pl.ds(s * EPT, CHUNK)],
                              rows_ref.at[buf], gs[buf]).wait()
        pltpu.make_async_copy(idx_hbm.at[s, 0], idxb_ref.at[buf], gs[buf]).wait()
        pltpu.async_copy(rows_ref.at[buf],
                         acc_ref.at[idxb_ref.at[buf, 0]], ss[buf], add=True)

        @pl.when(g >= 1)
        def _():
            pltpu.make_async_copy(rows_ref.at[b2],
                                  acc_ref.at[idxb_ref.at[b2, 0]],
                                  ss[b2]).wait()

        @pl.when(g + 2 < NCHUNK)
        def _():
            gather(g + 2, b2)

    gather(0, 0)
    gather(1, 1)
    plsc.subcore_barrier()

    def body(g3, carry):
        for u in range(3):
            g = 3 * g3 + u

            @pl.when(g < NCHUNK)
            def _():
                step(g, u)

        return carry

    lax.fori_loop(0, (NCHUNK + 2) // 3, body, 0)
    fb = (NCHUNK - 1) % 3
    pltpu.make_async_copy(rows_ref.at[fb],
                          acc_ref.at[idxb_ref.at[fb, 0]], ss[fb]).wait()
    plsc.subcore_barrier()
    pltpu.sync_copy(acc_ref.at[pl.ds(s * ROW_STRIDE, ROW_WIN)],
                    out_hbm.at[c, pl.ds(s * ROW_STRIDE, ROW_WIN)])


def _seg_stage(x_split, idj2d, zeros):
    mesh = plsc.VectorSubcoreMesh(core_axis_name="c", subcore_axis_name="s")
    fn = pl.kernel(
        _seg_body,
        out_type=jax.ShapeDtypeStruct((2, N_ATOMS, DH), jnp.float32),
        mesh=mesh,
        scratch_types=[
            pltpu.VMEM((3, 1, CHUNK), jnp.int32),
            pltpu.VMEM((3, CHUNK, DH), jnp.float32),
            pltpu.VMEM_SHARED((N_ATOMS, DH), jnp.float32),
            pltpu.SemaphoreType.DMA,
            pltpu.SemaphoreType.DMA,
            pltpu.SemaphoreType.DMA,
            pltpu.SemaphoreType.DMA,
            pltpu.SemaphoreType.DMA,
            pltpu.SemaphoreType.DMA,
        ],
    )
    return fn(x_split, idj2d, zeros)


def _mlp_body(x2_ref, w1_ref, wa0, wb0, wa1, wb1, wa2, wb2, o_ref):
    x2 = jnp.concatenate([x2_ref[0], x2_ref[1]], axis=-1)
    x = _silu(jnp.dot(x2, w1_ref[...], preferred_element_type=jnp.float32))
    for wa, wb in ((wa0, wb0), (wa1, wb1), (wa2, wb2)):
        y = _silu(jnp.dot(x, wa[...], preferred_element_type=jnp.float32))
        y = _silu(jnp.dot(y, wb[...], preferred_element_type=jnp.float32))
        x = (x + y) * INV_SQRT2
    o_ref[...] = x


def _mlp_stage(x2_split, W1, Wr0a, Wr0b, Wr1a, Wr1b, Wr2a, Wr2b):
    grid = (N_ATOMS // ATOM_BLK,)
    wspec = pl.BlockSpec((D_ATOM, D_ATOM), lambda i: (0, 0))
    return pl.pallas_call(
        _mlp_body,
        grid=grid,
        in_specs=[
            pl.BlockSpec((2, ATOM_BLK, DH), lambda i: (0, i, 0)),
            pl.BlockSpec((D_EDGE, D_ATOM), lambda i: (0, 0)),
            wspec, wspec, wspec, wspec, wspec, wspec,
        ],
        out_specs=pl.BlockSpec((ATOM_BLK, D_ATOM), lambda i: (i, 0)),
        out_shape=jax.ShapeDtypeStruct((N_ATOMS, D_ATOM), jnp.float32),
    )(x2_split, W1, Wr0a, Wr0b, Wr1a, Wr1b, Wr2a, Wr2b)


def kernel(h, m, rbf, id_j, W_rbf, W1, Wr0a, Wr0b, Wr1a, Wr1b, Wr2a, Wr2b):
    del h
    x_split = _edge_stage(m, rbf, W_rbf)
    idj4d = id_j.astype(jnp.int32).reshape(NS, NCHUNK, 1, CHUNK)
    zeros = jnp.zeros((ROW_WIN, DH), jnp.float32)
    x2_split = _seg_stage(x_split, idj4d, zeros)
    return _mlp_stage(x2_split, W1, Wr0a, Wr0b, Wr1a, Wr1b, Wr2a, Wr2b)


# speedup vs baseline: 2.3911x; 1.0538x over previous
import functools

import jax
import jax.numpy as jnp
import numpy as np
from jax import lax
from jax.experimental import pallas as pl
from jax.experimental.pallas import tpu as pltpu
from jax.experimental.pallas import tpu_sc as plsc

N_ATOMS = 10000
N_EDGES = 160000
D_ATOM = 512
D_EDGE = 256
D_RBF = 16
DH = D_EDGE // 2
INV_SQRT2 = np.float32(1.0 / np.sqrt(2.0))

NS = 16
SLICE_A = 79360
SLICE_B = N_EDGES - SLICE_A
CHUNK = 80
ROW_STRIDE = 624
ROW_WIN = 640

EDGE_BLK = 640
ATOM_BLK = 1000


def _silu(x):
    return x / (1.0 + jnp.exp(-x))


def _edge_body(m_ref, rbf_ref, wrbf_ref, o_ref):
    y = m_ref[...] * jnp.dot(rbf_ref[...], wrbf_ref[...],
                             preferred_element_type=jnp.float32)
    o_ref[0] = y[:, :DH]
    o_ref[1] = y[:, DH:]


def _edge_stage(m, rbf, W_rbf, blk0, nblk):
    return pl.pallas_call(
        _edge_body,
        grid=(nblk,),
        in_specs=[
            pl.BlockSpec((EDGE_BLK, D_EDGE), lambda i: (blk0 + i, 0)),
            pl.BlockSpec((EDGE_BLK, D_RBF), lambda i: (blk0 + i, 0)),
            pl.BlockSpec((D_RBF, D_EDGE), lambda i: (0, 0)),
        ],
        out_specs=pl.BlockSpec((2, EDGE_BLK, DH), lambda i: (0, i, 0)),
        out_shape=jax.ShapeDtypeStruct((2, nblk * EDGE_BLK, DH), jnp.float32),
    )(m, rbf, W_rbf)


def _seg_body(ept, nchunk, x_hbm, idx_hbm, init_hbm, out_hbm,
              idxb_ref, rows_ref, acc_ref, gs0, gs1, gs2, ss0, ss1, ss2):
    c = lax.axis_index("c")
    s = lax.axis_index("s")
    gs = (gs0, gs1, gs2)
    ss = (ss0, ss1, ss2)
    pltpu.sync_copy(init_hbm.at[c, pl.ds(s * ROW_STRIDE, ROW_WIN)],
                    acc_ref.at[pl.ds(s * ROW_STRIDE, ROW_WIN)])

    def gather(grp, buf):
        e0 = s * ept + grp * CHUNK
        pltpu.async_copy(x_hbm.at[c, pl.ds(e0, CHUNK)],
                         rows_ref.at[buf], gs[buf])
        pltpu.async_copy(idx_hbm.at[s, grp], idxb_ref.at[buf], gs[buf])

    def step(g, buf):
        b2 = (buf + 2) % 3
        pltpu.make_async_copy(x_hbm.at[c, pl.ds(s * ept, CHUNK)],
                              rows_ref.at[buf], gs[buf]).wait()
        pltpu.make_async_copy(idx_hbm.at[s, 0], idxb_ref.at[buf], gs[buf]).wait()
        pltpu.async_copy(rows_ref.at[buf],
                         acc_ref.at[idxb_ref.at[buf, 0]], ss[buf], add=True)

        @pl.when(g >= 1)
        def _():
            pltpu.make_async_copy(rows_ref.at[b2],
                                  acc_ref.at[idxb_ref.at[b2, 0]],
                                  ss[b2]).wait()

        @pl.when(g + 2 < nchunk)
        def _():
            gather(g + 2, b2)

    gather(0, 0)
    gather(1, 1)
    plsc.subcore_barrier()

    def body(g3, carry):
        for u in range(3):
            g = 3 * g3 + u

            @pl.when(g < nchunk)
            def _():
                step(g, u)

        return carry

    lax.fori_loop(0, (nchunk + 2) // 3, body, 0)
    fb = (nchunk - 1) % 3
    pltpu.make_async_copy(rows_ref.at[fb],
                          acc_ref.at[idxb_ref.at[fb, 0]], ss[fb]).wait()
    plsc.subcore_barrier()
    pltpu.sync_copy(acc_ref.at[pl.ds(s * ROW_STRIDE, ROW_WIN)],
                    out_hbm.at[c, pl.ds(s * ROW_STRIDE, ROW_WIN)])


def _seg_stage(x_split, idj4d, init, ept, nchunk):
    mesh = plsc.VectorSubcoreMesh(core_axis_name="c", subcore_axis_name="s")
    fn = pl.kernel(
        functools.partial(_seg_body, ept, nchunk),
        out_type=jax.ShapeDtypeStruct((2, N_ATOMS, DH), jnp.float32),
        mesh=mesh,
        scratch_types=[
            pltpu.VMEM((3, 1, CHUNK), jnp.int32),
            pltpu.VMEM((3, CHUNK, DH), jnp.float32),
            pltpu.VMEM_SHARED((N_ATOMS, DH), jnp.float32),
            pltpu.SemaphoreType.DMA,
            pltpu.SemaphoreType.DMA,
            pltpu.SemaphoreType.DMA,
            pltpu.SemaphoreType.DMA,
            pltpu.SemaphoreType.DMA,
            pltpu.SemaphoreType.DMA,
        ],
    )
    return fn(x_split, idj4d, init)


def _mlp_body(x2_ref, w1_ref, wa0, wb0, wa1, wb1, wa2, wb2, o_ref):
    x2 = jnp.concatenate([x2_ref[0], x2_ref[1]], axis=-1)
    x = _silu(jnp.dot(x2, w1_ref[...], preferred_element_type=jnp.float32))
    for wa, wb in ((wa0, wb0), (wa1, wb1), (wa2, wb2)):
        y = _silu(jnp.dot(x, wa[...], preferred_element_type=jnp.float32))
        y = _silu(jnp.dot(y, wb[...], preferred_element_type=jnp.float32))
        x = (x + y) * INV_SQRT2
    o_ref[...] = x


def _mlp_stage(x2_split, W1, Wr0a, Wr0b, Wr1a, Wr1b, Wr2a, Wr2b):
    grid = (N_ATOMS // ATOM_BLK,)
    wspec = pl.BlockSpec((D_ATOM, D_ATOM), lambda i: (0, 0))
    return pl.pallas_call(
        _mlp_body,
        grid=grid,
        in_specs=[
            pl.BlockSpec((2, ATOM_BLK, DH), lambda i: (0, i, 0)),
            pl.BlockSpec((D_EDGE, D_ATOM), lambda i: (0, 0)),
            wspec, wspec, wspec, wspec, wspec, wspec,
        ],
        out_specs=pl.BlockSpec((ATOM_BLK, D_ATOM), lambda i: (i, 0)),
        out_shape=jax.ShapeDtypeStruct((N_ATOMS, D_ATOM), jnp.float32),
    )(x2_split, W1, Wr0a, Wr0b, Wr1a, Wr1b, Wr2a, Wr2b)


def kernel(h, m, rbf, id_j, W_rbf, W1, Wr0a, Wr0b, Wr1a, Wr1b, Wr2a, Wr2b):
    del h
    blk_a = SLICE_A // EDGE_BLK
    blk_b = SLICE_B // EDGE_BLK
    xa = _edge_stage(m, rbf, W_rbf, 0, blk_a)
    xb = _edge_stage(m, rbf, W_rbf, blk_a, blk_b)
    idj = id_j.astype(jnp.int32)
    ia = idj[:SLICE_A].reshape(NS, SLICE_A // (NS * CHUNK), 1, CHUNK)
    ib = idj[SLICE_A:].reshape(NS, SLICE_B // (NS * CHUNK), 1, CHUNK)
    zeros = jnp.zeros((2, N_ATOMS, DH), jnp.float32)
    pa = _seg_stage(xa, ia, zeros, SLICE_A // NS, SLICE_A // (NS * CHUNK))
    x2_split = _seg_stage(xb, ib, pa, SLICE_B // NS, SLICE_B // (NS * CHUNK))
    return _mlp_stage(x2_split, W1, Wr0a, Wr0b, Wr1a, Wr1b, Wr2a, Wr2b)


# speedup vs baseline: 2.7548x; 1.1521x over previous
import functools

import jax
import jax.numpy as jnp
import numpy as np
from jax import lax
from jax.experimental import pallas as pl
from jax.experimental.pallas import tpu as pltpu
from jax.experimental.pallas import tpu_sc as plsc

N_ATOMS = 10000
N_EDGES = 160000
D_ATOM = 512
D_EDGE = 256
D_RBF = 16
DH = D_EDGE // 2
INV_SQRT2 = np.float32(1.0 / np.sqrt(2.0))

NS = 16
SLICE_A = 106240
SLICE_B = N_EDGES - SLICE_A
CHUNK = 80
ROW_STRIDE = 624
ROW_WIN = 640

EDGE_BLK = 1280
ATOM_BLK = 2000


def _silu(x):
    return x / (1.0 + jnp.exp(-x))


def _edge_body(m_ref, rbf_ref, wrbf_ref, o_ref):
    y = m_ref[...] * jnp.dot(rbf_ref[...], wrbf_ref[...],
                             preferred_element_type=jnp.float32)
    o_ref[0] = y[:, :DH]
    o_ref[1] = y[:, DH:]


def _edge_stage(m, rbf, W_rbf, blk0, nblk):
    return pl.pallas_call(
        _edge_body,
        grid=(nblk,),
        in_specs=[
            pl.BlockSpec((EDGE_BLK, D_EDGE), lambda i: (blk0 + i, 0)),
            pl.BlockSpec((EDGE_BLK, D_RBF), lambda i: (blk0 + i, 0)),
            pl.BlockSpec((D_RBF, D_EDGE), lambda i: (0, 0)),
        ],
        out_specs=pl.BlockSpec((2, EDGE_BLK, DH), lambda i: (0, i, 0)),
        out_shape=jax.ShapeDtypeStruct((2, nblk * EDGE_BLK, DH), jnp.float32),
    )(m, rbf, W_rbf)


def _seg_body(ept, nchunk, x_hbm, idx_hbm, init_hbm, out_hbm,
              idxb_ref, rows_ref, acc_ref, gs0, gs1, gs2, ss0, ss1, ss2):
    c = lax.axis_index("c")
    s = lax.axis_index("s")
    gs = (gs0, gs1, gs2)
    ss = (ss0, ss1, ss2)
    pltpu.sync_copy(init_hbm.at[c, pl.ds(s * ROW_STRIDE, ROW_WIN)],
                    acc_ref.at[pl.ds(s * ROW_STRIDE, ROW_WIN)])

    def gather(grp, buf):
        e0 = s * ept + grp * CHUNK
        pltpu.async_copy(x_hbm.at[c, pl.ds(e0, CHUNK)],
                         rows_ref.at[buf], gs[buf])
        pltpu.async_copy(idx_hbm.at[s, grp], idxb_ref.at[buf], gs[buf])

    def step(g, buf):
        b2 = (buf + 2) % 3
        pltpu.make_async_copy(x_hbm.at[c, pl.ds(s * ept, CHUNK)],
                              rows_ref.at[buf], gs[buf]).wait()
        pltpu.make_async_copy(idx_hbm.at[s, 0], idxb_ref.at[buf], gs[buf]).wait()
        pltpu.async_copy(rows_ref.at[buf],
                         acc_ref.at[idxb_ref.at[buf, 0]], ss[buf], add=True)

        @pl.when(g >= 1)
        def _():
            pltpu.make_async_copy(rows_ref.at[b2],
                                  acc_ref.at[idxb_ref.at[b2, 0]],
                                  ss[b2]).wait()

        @pl.when(g + 2 < nchunk)
        def _():
            gather(g + 2, b2)

    gather(0, 0)
    gather(1, 1)
    plsc.subcore_barrier()

    def body(g3, carry):
        for u in range(3):
            g = 3 * g3 + u

            @pl.when(g < nchunk)
            def _():
                step(g, u)

        return carry

    lax.fori_loop(0, (nchunk + 2) // 3, body, 0)
    fb = (nchunk - 1) % 3
    pltpu.make_async_copy(rows_ref.at[fb],
                          acc_ref.at[idxb_ref.at[fb, 0]], ss[fb]).wait()
    plsc.subcore_barrier()
    pltpu.sync_copy(acc_ref.at[pl.ds(s * ROW_STRIDE, ROW_WIN)],
                    out_hbm.at[c, pl.ds(s * ROW_STRIDE, ROW_WIN)])


def _seg_stage(x_split, idj4d, init, ept, nchunk):
    mesh = plsc.VectorSubcoreMesh(core_axis_name="c", subcore_axis_name="s")
    fn = pl.kernel(
        functools.partial(_seg_body, ept, nchunk),
        out_type=jax.ShapeDtypeStruct((2, N_ATOMS, DH), jnp.float32),
        mesh=mesh,
        scratch_types=[
            pltpu.VMEM((3, 1, CHUNK), jnp.int32),
            pltpu.VMEM((3, CHUNK, DH), jnp.float32),
            pltpu.VMEM_SHARED((N_ATOMS, DH), jnp.float32),
            pltpu.SemaphoreType.DMA,
            pltpu.SemaphoreType.DMA,
            pltpu.SemaphoreType.DMA,
            pltpu.SemaphoreType.DMA,
            pltpu.SemaphoreType.DMA,
            pltpu.SemaphoreType.DMA,
        ],
    )
    return fn(x_split, idj4d, init)


def _mlp_body(x2_ref, w1_ref, wa0, wb0, wa1, wb1, wa2, wb2, o_ref):
    x2 = jnp.concatenate([x2_ref[0], x2_ref[1]], axis=-1)
    x = _silu(jnp.dot(x2, w1_ref[...], preferred_element_type=jnp.float32))
    for wa, wb in ((wa0, wb0), (wa1, wb1), (wa2, wb2)):
        y = _silu(jnp.dot(x, wa[...], preferred_element_type=jnp.float32))
        y = _silu(jnp.dot(y, wb[...], preferred_element_type=jnp.float32))
        x = (x + y) * INV_SQRT2
    o_ref[...] = x


def _mlp_stage(x2_split, W1, Wr0a, Wr0b, Wr1a, Wr1b, Wr2a, Wr2b):
    grid = (N_ATOMS // ATOM_BLK,)
    wspec = pl.BlockSpec((D_ATOM, D_ATOM), lambda i: (0, 0))
    return pl.pallas_call(
        _mlp_body,
        grid=grid,
        in_specs=[
            pl.BlockSpec((2, ATOM_BLK, DH), lambda i: (0, i, 0)),
            pl.BlockSpec((D_EDGE, D_ATOM), lambda i: (0, 0)),
            wspec, wspec, wspec, wspec, wspec, wspec,
        ],
        out_specs=pl.BlockSpec((ATOM_BLK, D_ATOM), lambda i: (i, 0)),
        out_shape=jax.ShapeDtypeStruct((N_ATOMS, D_ATOM), jnp.float32),
    )(x2_split, W1, Wr0a, Wr0b, Wr1a, Wr1b, Wr2a, Wr2b)


def kernel(h, m, rbf, id_j, W_rbf, W1, Wr0a, Wr0b, Wr1a, Wr1b, Wr2a, Wr2b):
    del h
    blk_a = SLICE_A // EDGE_BLK
    blk_b = SLICE_B // EDGE_BLK
    xa = _edge_stage(m, rbf, W_rbf, 0, blk_a)
    xb = _edge_stage(m, rbf, W_rbf, blk_a, blk_b)
    idj = id_j.astype(jnp.int32)
    ia = idj[:SLICE_A].reshape(NS, SLICE_A // (NS * CHUNK), 1, CHUNK)
    ib = idj[SLICE_A:].reshape(NS, SLICE_B // (NS * CHUNK), 1, CHUNK)
    zeros = jnp.zeros((2, N_ATOMS, DH), jnp.float32)
    pa = _seg_stage(xa, ia, zeros, SLICE_A // NS, SLICE_A // (NS * CHUNK))
    x2_split = _seg_stage(xb, ib, pa, SLICE_B // NS, SLICE_B // (NS * CHUNK))
    return _mlp_stage(x2_split, W1, Wr0a, Wr0b, Wr1a, Wr1b, Wr2a, Wr2b)


# speedup vs baseline: 2.7818x; 1.0098x over previous
import functools

import jax
import jax.numpy as jnp
import numpy as np
from jax import lax
from jax.experimental import pallas as pl
from jax.experimental.pallas import tpu as pltpu
from jax.experimental.pallas import tpu_sc as plsc

N_ATOMS = 10000
N_EDGES = 160000
D_ATOM = 512
D_EDGE = 256
D_RBF = 16
DH = D_EDGE // 2
INV_SQRT2 = np.float32(1.0 / np.sqrt(2.0))

NS = 16
SLICE_A = 99840
SLICE_B = N_EDGES - SLICE_A
CHUNK = 80
ROW_STRIDE = 624
ROW_WIN = 640

EDGE_BLK = 1280
ATOM_BLK = 2000


def _silu(x):
    return x / (1.0 + jnp.exp(-x))


def _edge_body(m_ref, rbf_ref, wrbf_ref, o_ref):
    y = m_ref[...] * jnp.dot(rbf_ref[...], wrbf_ref[...],
                             preferred_element_type=jnp.float32)
    o_ref[0] = y[:, :DH]
    o_ref[1] = y[:, DH:]


def _edge_stage(m, rbf, W_rbf, blk0, nblk):
    return pl.pallas_call(
        _edge_body,
        grid=(nblk,),
        in_specs=[
            pl.BlockSpec((EDGE_BLK, D_EDGE), lambda i: (blk0 + i, 0)),
            pl.BlockSpec((EDGE_BLK, D_RBF), lambda i: (blk0 + i, 0)),
            pl.BlockSpec((D_RBF, D_EDGE), lambda i: (0, 0)),
        ],
        out_specs=pl.BlockSpec((2, EDGE_BLK, DH), lambda i: (0, i, 0)),
        out_shape=jax.ShapeDtypeStruct((2, nblk * EDGE_BLK, DH), jnp.float32),
    )(m, rbf, W_rbf)


def _seg_body(ept, nchunk, x_hbm, idx_hbm, init_hbm, out_hbm,
              idxb_ref, rows_ref, acc_ref, gs0, gs1, gs2, ss0, ss1, ss2):
    c = lax.axis_index("c")
    s = lax.axis_index("s")
    gs = (gs0, gs1, gs2)
    ss = (ss0, ss1, ss2)
    pltpu.sync_copy(init_hbm.at[c, pl.ds(s * ROW_STRIDE, ROW_WIN)],
                    acc_ref.at[pl.ds(s * ROW_STRIDE, ROW_WIN)])

    def gather(grp, buf):
        e0 = s * ept + grp * CHUNK
        pltpu.async_copy(x_hbm.at[c, pl.ds(e0, CHUNK)],
                         rows_ref.at[buf], gs[buf])
        pltpu.async_copy(idx_hbm.at[s, grp], idxb_ref.at[buf], gs[buf])

    def step(g, buf):
        b2 = (buf + 2) % 3
        pltpu.make_async_copy(x_hbm.at[c, pl.ds(s * ept, CHUNK)],
                              rows_ref.at[buf], gs[buf]).wait()
        pltpu.make_async_copy(idx_hbm.at[s, 0], idxb_ref.at[buf], gs[buf]).wait()
        pltpu.async_copy(rows_ref.at[buf],
                         acc_ref.at[idxb_ref.at[buf, 0]], ss[buf], add=True)

        @pl.when(g >= 1)
        def _():
            pltpu.make_async_copy(rows_ref.at[b2],
                                  acc_ref.at[idxb_ref.at[b2, 0]],
                                  ss[b2]).wait()

        @pl.when(g + 2 < nchunk)
        def _():
            gather(g + 2, b2)

    gather(0, 0)
    gather(1, 1)
    plsc.subcore_barrier()

    def body(g3, carry):
        for u in range(3):
            g = 3 * g3 + u

            @pl.when(g < nchunk)
            def _():
                step(g, u)

        return carry

    lax.fori_loop(0, (nchunk + 2) // 3, body, 0)
    fb = (nchunk - 1) % 3
    pltpu.make_async_copy(rows_ref.at[fb],
                          acc_ref.at[idxb_ref.at[fb, 0]], ss[fb]).wait()
    plsc.subcore_barrier()
    pltpu.sync_copy(acc_ref.at[pl.ds(s * ROW_STRIDE, ROW_WIN)],
                    out_hbm.at[c, pl.ds(s * ROW_STRIDE, ROW_WIN)])


def _seg_stage(x_split, idj4d, init, ept, nchunk):
    mesh = plsc.VectorSubcoreMesh(core_axis_name="c", subcore_axis_name="s")
    fn = pl.kernel(
        functools.partial(_seg_body, ept, nchunk),
        out_type=jax.ShapeDtypeStruct((2, N_ATOMS, DH), jnp.float32),
        mesh=mesh,
        scratch_types=[
            pltpu.VMEM((3, 1, CHUNK), jnp.int32),
            pltpu.VMEM((3, CHUNK, DH), jnp.float32),
            pltpu.VMEM_SHARED((N_ATOMS, DH), jnp.float32),
            pltpu.SemaphoreType.DMA,
            pltpu.SemaphoreType.DMA,
            pltpu.SemaphoreType.DMA,
            pltpu.SemaphoreType.DMA,
            pltpu.SemaphoreType.DMA,
            pltpu.SemaphoreType.DMA,
        ],
    )
    return fn(x_split, idj4d, init)


def _mlp_body(x2_ref, w1_ref, wa0, wb0, wa1, wb1, wa2, wb2, o_ref):
    x2 = jnp.concatenate([x2_ref[0], x2_ref[1]], axis=-1)
    x = _silu(jnp.dot(x2, w1_ref[...], preferred_element_type=jnp.float32))
    for wa, wb in ((wa0, wb0), (wa1, wb1), (wa2, wb2)):
        y = _silu(jnp.dot(x, wa[...], preferred_element_type=jnp.float32))
        y = _silu(jnp.dot(y, wb[...], preferred_element_type=jnp.float32))
        x = (x + y) * INV_SQRT2
    o_ref[...] = x


def _mlp_stage(x2_split, W1, Wr0a, Wr0b, Wr1a, Wr1b, Wr2a, Wr2b):
    grid = (N_ATOMS // ATOM_BLK,)
    wspec = pl.BlockSpec((D_ATOM, D_ATOM), lambda i: (0, 0))
    return pl.pallas_call(
        _mlp_body,
        grid=grid,
        in_specs=[
            pl.BlockSpec((2, ATOM_BLK, DH), lambda i: (0, i, 0)),
            pl.BlockSpec((D_EDGE, D_ATOM), lambda i: (0, 0)),
            wspec, wspec, wspec, wspec, wspec, wspec,
        ],
        out_specs=pl.BlockSpec((ATOM_BLK, D_ATOM), lambda i: (i, 0)),
        out_shape=jax.ShapeDtypeStruct((N_ATOMS, D_ATOM), jnp.float32),
    )(x2_split, W1, Wr0a, Wr0b, Wr1a, Wr1b, Wr2a, Wr2b)


def kernel(h, m, rbf, id_j, W_rbf, W1, Wr0a, Wr0b, Wr1a, Wr1b, Wr2a, Wr2b):
    del h
    blk_a = SLICE_A // EDGE_BLK
    blk_b = SLICE_B // EDGE_BLK
    xa = _edge_stage(m, rbf, W_rbf, 0, blk_a)
    xb = _edge_stage(m, rbf, W_rbf, blk_a, blk_b)
    idj = id_j.astype(jnp.int32)
    ia = idj[:SLICE_A].reshape(NS, SLICE_A // (NS * CHUNK), 1, CHUNK)
    ib = idj[SLICE_A:].reshape(NS, SLICE_B // (NS * CHUNK), 1, CHUNK)
    zeros = jnp.zeros((2, N_ATOMS, DH), jnp.float32)
    pa = _seg_stage(xa, ia, zeros, SLICE_A // NS, SLICE_A // (NS * CHUNK))
    x2_split = _seg_stage(xb, ib, pa, SLICE_B // NS, SLICE_B // (NS * CHUNK))
    return _mlp_stage(x2_split, W1, Wr0a, Wr0b, Wr1a, Wr1b, Wr2a, Wr2b)


# speedup vs baseline: 2.8590x; 1.0278x over previous
import functools

import jax
import jax.numpy as jnp
import numpy as np
from jax import lax
from jax.experimental import pallas as pl
from jax.experimental.pallas import tpu as pltpu
from jax.experimental.pallas import tpu_sc as plsc

N_ATOMS = 10000
N_EDGES = 160000
D_ATOM = 512
D_EDGE = 256
D_RBF = 16
DH = D_EDGE // 2
INV_SQRT2 = np.float32(1.0 / np.sqrt(2.0))

NS = 16
SLICE_A = 96000
SLICE_B = N_EDGES - SLICE_A
CHUNK = 80
ROW_STRIDE = 624
ROW_WIN = 640

EDGE_BLK = 1600
ATOM_BLK = 2000


def _silu(x):
    return x / (1.0 + jnp.exp(-x))


def _edge_body(m_ref, rbf_ref, wrbf_ref, o_ref):
    y = m_ref[...] * jnp.dot(rbf_ref[...], wrbf_ref[...],
                             preferred_element_type=jnp.float32)
    o_ref[0] = y[:, :DH]
    o_ref[1] = y[:, DH:]


def _edge_stage(m, rbf, W_rbf, blk0, nblk):
    return pl.pallas_call(
        _edge_body,
        grid=(nblk,),
        in_specs=[
            pl.BlockSpec((EDGE_BLK, D_EDGE), lambda i: (blk0 + i, 0)),
            pl.BlockSpec((EDGE_BLK, D_RBF), lambda i: (blk0 + i, 0)),
            pl.BlockSpec((D_RBF, D_EDGE), lambda i: (0, 0)),
        ],
        out_specs=pl.BlockSpec((2, EDGE_BLK, DH), lambda i: (0, i, 0)),
        out_shape=jax.ShapeDtypeStruct((2, nblk * EDGE_BLK, DH), jnp.float32),
    )(m, rbf, W_rbf)


def _seg_body(ept, nchunk, first, x_hbm, idx_hbm, init_hbm, out_hbm,
              idxb_ref, rows_ref, acc_ref, gs0, gs1, gs2, ss0, ss1, ss2):
    c = lax.axis_index("c")
    s = lax.axis_index("s")
    gs = (gs0, gs1, gs2)
    ss = (ss0, ss1, ss2)
    if first:
        pltpu.sync_copy(init_hbm, acc_ref.at[pl.ds(s * ROW_STRIDE, ROW_WIN)])
    else:
        pltpu.sync_copy(init_hbm.at[c, pl.ds(s * ROW_STRIDE, ROW_WIN)],
                        acc_ref.at[pl.ds(s * ROW_STRIDE, ROW_WIN)])

    def gather(grp, buf):
        e0 = s * ept + grp * CHUNK
        pltpu.async_copy(x_hbm.at[c, pl.ds(e0, CHUNK)],
                         rows_ref.at[buf], gs[buf])
        pltpu.async_copy(idx_hbm.at[s, grp], idxb_ref.at[buf], gs[buf])

    def step(g, buf):
        b2 = (buf + 2) % 3
        pltpu.make_async_copy(x_hbm.at[c, pl.ds(s * ept, CHUNK)],
                              rows_ref.at[buf], gs[buf]).wait()
        pltpu.make_async_copy(idx_hbm.at[s, 0], idxb_ref.at[buf], gs[buf]).wait()
        pltpu.async_copy(rows_ref.at[buf],
                         acc_ref.at[idxb_ref.at[buf, 0]], ss[buf], add=True)

        @pl.when(g >= 1)
        def _():
            pltpu.make_async_copy(rows_ref.at[b2],
                                  acc_ref.at[idxb_ref.at[b2, 0]],
                                  ss[b2]).wait()

        @pl.when(g + 2 < nchunk)
        def _():
            gather(g + 2, b2)

    gather(0, 0)
    gather(1, 1)
    plsc.subcore_barrier()

    def body(g3, carry):
        for u in range(3):
            g = 3 * g3 + u

            @pl.when(g < nchunk)
            def _():
                step(g, u)

        return carry

    lax.fori_loop(0, (nchunk + 2) // 3, body, 0)
    fb = (nchunk - 1) % 3
    pltpu.make_async_copy(rows_ref.at[fb],
                          acc_ref.at[idxb_ref.at[fb, 0]], ss[fb]).wait()
    plsc.subcore_barrier()
    pltpu.sync_copy(acc_ref.at[pl.ds(s * ROW_STRIDE, ROW_WIN)],
                    out_hbm.at[c, pl.ds(s * ROW_STRIDE, ROW_WIN)])


def _seg_stage(x_split, idj4d, init, ept, nchunk, first=False):
    mesh = plsc.VectorSubcoreMesh(core_axis_name="c", subcore_axis_name="s")
    fn = pl.kernel(
        functools.partial(_seg_body, ept, nchunk, first),
        out_type=jax.ShapeDtypeStruct((2, N_ATOMS, DH), jnp.float32),
        mesh=mesh,
        scratch_types=[
            pltpu.VMEM((3, 1, CHUNK), jnp.int32),
            pltpu.VMEM((3, CHUNK, DH), jnp.float32),
            pltpu.VMEM_SHARED((N_ATOMS, DH), jnp.float32),
            pltpu.SemaphoreType.DMA,
            pltpu.SemaphoreType.DMA,
            pltpu.SemaphoreType.DMA,
            pltpu.SemaphoreType.DMA,
            pltpu.SemaphoreType.DMA,
            pltpu.SemaphoreType.DMA,
        ],
    )
    return fn(x_split, idj4d, init)


def _mlp_body(x2_ref, w1_ref, wa0, wb0, wa1, wb1, wa2, wb2, o_ref):
    x2 = jnp.concatenate([x2_ref[0], x2_ref[1]], axis=-1)
    x = _silu(jnp.dot(x2, w1_ref[...], preferred_element_type=jnp.float32))
    for wa, wb in ((wa0, wb0), (wa1, wb1), (wa2, wb2)):
        y = _silu(jnp.dot(x, wa[...], preferred_element_type=jnp.float32))
        y = _silu(jnp.dot(y, wb[...], preferred_element_type=jnp.float32))
        x = (x + y) * INV_SQRT2
    o_ref[...] = x


def _mlp_stage(x2_split, W1, Wr0a, Wr0b, Wr1a, Wr1b, Wr2a, Wr2b):
    grid = (N_ATOMS // ATOM_BLK,)
    wspec = pl.BlockSpec((D_ATOM, D_ATOM), lambda i: (0, 0))
    return pl.pallas_call(
        _mlp_body,
        grid=grid,
        in_specs=[
            pl.BlockSpec((2, ATOM_BLK, DH), lambda i: (0, i, 0)),
            pl.BlockSpec((D_EDGE, D_ATOM), lambda i: (0, 0)),
            wspec, wspec, wspec, wspec, wspec, wspec,
        ],
        out_specs=pl.BlockSpec((ATOM_BLK, D_ATOM), lambda i: (i, 0)),
        out_shape=jax.ShapeDtypeStruct((N_ATOMS, D_ATOM), jnp.float32),
    )(x2_split, W1, Wr0a, Wr0b, Wr1a, Wr1b, Wr2a, Wr2b)


def kernel(h, m, rbf, id_j, W_rbf, W1, Wr0a, Wr0b, Wr1a, Wr1b, Wr2a, Wr2b):
    del h
    blk_a = SLICE_A // EDGE_BLK
    blk_b = SLICE_B // EDGE_BLK
    xa = _edge_stage(m, rbf, W_rbf, 0, blk_a)
    xb = _edge_stage(m, rbf, W_rbf, blk_a, blk_b)
    idj = id_j.astype(jnp.int32)
    ia = idj[:SLICE_A].reshape(NS, SLICE_A // (NS * CHUNK), 1, CHUNK)
    ib = idj[SLICE_A:].reshape(NS, SLICE_B // (NS * CHUNK), 1, CHUNK)
    zeros = jnp.zeros((ROW_WIN, DH), jnp.float32)
    pa = _seg_stage(xa, ia, zeros, SLICE_A // NS, SLICE_A // (NS * CHUNK),
                    first=True)
    x2_split = _seg_stage(xb, ib, pa, SLICE_B // NS, SLICE_B // (NS * CHUNK))
    return _mlp_stage(x2_split, W1, Wr0a, Wr0b, Wr1a, Wr1b, Wr2a, Wr2b)
